# baseline (speedup 1.0000x reference)
import functools

import jax
import jax.numpy as jnp
from jax import lax
from jax.experimental import pallas as pl
from jax.experimental.pallas import tpu as pltpu
from jax.experimental.pallas import tpu_sc as plsc

LANES = 16


@functools.lru_cache(maxsize=None)
def _build(batch: int):
    info = plsc.get_sparse_core_info()
    nc, ns = 1, info.num_subcores
    nw = nc * ns
    assert batch % (8 * nw) == 0, "HBM 1-D slice offsets must be 8-aligned"
    b_per_w = batch // nw

    mesh = plsc.VectorSubcoreMesh(
        core_axis_name="c", subcore_axis_name="s", num_cores=nc
    )

    @functools.partial(
        pl.kernel,
        mesh=mesh,
        out_type=jax.ShapeDtypeStruct((batch,), jnp.float32),
        compiler_params=pltpu.CompilerParams(needs_layout_passes=False),
        scratch_types=[
            pltpu.VMEM((b_per_w,), jnp.int32),
            pltpu.VMEM((b_per_w,), jnp.float32),
            pltpu.VMEM((3, 4), jnp.float32),
            pltpu.VMEM((4, 1), jnp.float32),
            pltpu.VMEM((1,), jnp.float32),
            pltpu.VMEM((LANES,), jnp.float32),
            pltpu.SemaphoreType.DMA,
            pltpu.SemaphoreType.DMA,
            pltpu.SemaphoreType.DMA,
            pltpu.SemaphoreType.DMA,
            pltpu.SemaphoreType.DMA,
        ],
    )
    def sc_kernel(idx_hbm, emb_hbm, w_hbm, b_hbm, out_hbm, idx_v, out_v, e_v, w_v, b_v, scr_v, sem_i0, sem_i1, sem_w, sem_o0, sem_o1):
        wid = lax.axis_index("s") * nc + lax.axis_index("c")
        base = wid * b_per_w
        half = b_per_w // 2
        idx_cp0 = pltpu.async_copy(
            idx_hbm.at[pl.ds(base, half)], idx_v.at[pl.ds(0, half)], sem_i0
        )
        idx_cp1 = pltpu.async_copy(
            idx_hbm.at[pl.ds(base + half, half)], idx_v.at[pl.ds(half, half)], sem_i1
        )
        w_cps = [
            pltpu.async_copy(emb_hbm, e_v, sem_w),
            pltpu.async_copy(w_hbm, w_v, sem_w),
            pltpu.async_copy(b_hbm, b_v, sem_w),
        ]
        for cp in w_cps:
            cp.wait()

        lane = lax.iota(jnp.int32, LANES)
        zero = jnp.zeros((LANES,), jnp.float32)
        zero_i = jnp.zeros((LANES,), jnp.int32)
        row = jnp.minimum(lane >> 2, jnp.full((LANES,), 2, jnp.int32))
        col = lane & jnp.full((LANES,), 3, jnp.int32)
        valid = lane < 12
        ep = jnp.where(valid, plsc.load_gather(e_v, [row, col]), zero)
        wp = jnp.where(valid, plsc.load_gather(w_v, [col, zero_i]), zero)
        bias = plsc.load_gather(b_v, [zero_i])

        scr_v[...] = ep * wp
        seg = lane & jnp.full((LANES,), -4, jnp.int32)
        one_i = jnp.ones((LANES,), jnp.int32)
        sum4 = (
            plsc.load_gather(scr_v, [seg])
            + plsc.load_gather(scr_v, [seg + one_i])
            + plsc.load_gather(scr_v, [seg + 2 * one_i])
            + plsc.load_gather(scr_v, [seg + 3 * one_i])
        )
        one = jnp.ones((LANES,), jnp.float32)
        sig = one / (one + jnp.exp(-(sum4 + bias)))
        scr_v[...] = sig
        table_idx = jnp.minimum(lane * 4, jnp.full((LANES,), 8, jnp.int32))
        scr_v[...] = plsc.load_gather(scr_v, [table_idx])

        def body(j, carry):
            o = j * LANES
            iv = idx_v[pl.ds(o, LANES)]
            out_v[pl.ds(o, LANES)] = plsc.load_gather(scr_v, [iv])
            return carry

        idx_cp0.wait()
        lax.fori_loop(0, half // LANES, body, 0)
        out_cp0 = pltpu.async_copy(
            out_v.at[pl.ds(0, half)], out_hbm.at[pl.ds(base, half)], sem_o0
        )
        idx_cp1.wait()
        lax.fori_loop(half // LANES, b_per_w // LANES, body, 0)
        out_cp1 = pltpu.async_copy(
            out_v.at[pl.ds(half, half)], out_hbm.at[pl.ds(base + half, half)], sem_o1
        )
        out_cp0.wait()
        out_cp1.wait()

    return sc_kernel


def kernel(indices, emb_table, W, b):
    out = _build(indices.shape[0])(indices, emb_table, W, b)
    return out.reshape(-1, 1)


# speedup vs baseline: 1.0335x; 1.0335x over previous
import functools

import jax
import jax.numpy as jnp
from jax import lax
from jax.experimental import pallas as pl
from jax.experimental.pallas import tpu as pltpu
from jax.experimental.pallas import tpu_sc as plsc

LANES = 16


@functools.lru_cache(maxsize=None)
def _build(batch: int):
    info = plsc.get_sparse_core_info()
    nc, ns = 1, info.num_subcores
    nw = nc * ns
    assert batch % (8 * nw) == 0, "HBM 1-D slice offsets must be 8-aligned"
    b_per_w = batch // nw

    mesh = plsc.VectorSubcoreMesh(
        core_axis_name="c", subcore_axis_name="s", num_cores=nc
    )

    @functools.partial(
        pl.kernel,
        mesh=mesh,
        out_type=jax.ShapeDtypeStruct((batch,), jnp.float32),
        compiler_params=pltpu.CompilerParams(
            needs_layout_passes=False, use_tc_tiling_on_sc=False
        ),
        scratch_types=[
            pltpu.VMEM((b_per_w,), jnp.int32),
            pltpu.VMEM((b_per_w,), jnp.float32),
            pltpu.VMEM((3, 4), jnp.float32),
            pltpu.VMEM((4, 1), jnp.float32),
            pltpu.VMEM((1,), jnp.float32),
            pltpu.VMEM((LANES,), jnp.float32),
            pltpu.SemaphoreType.DMA,
            pltpu.SemaphoreType.DMA,
            pltpu.SemaphoreType.DMA,
            pltpu.SemaphoreType.DMA,
            pltpu.SemaphoreType.DMA,
        ],
    )
    def sc_kernel(idx_hbm, emb_hbm, w_hbm, b_hbm, out_hbm, idx_v, out_v, e_v, w_v, b_v, scr_v, sem_i0, sem_i1, sem_w, sem_o0, sem_o1):
        wid = lax.axis_index("s") * nc + lax.axis_index("c")
        base = wid * b_per_w
        half = b_per_w // 2
        idx_cp0 = pltpu.async_copy(
            idx_hbm.at[pl.ds(base, half)], idx_v.at[pl.ds(0, half)], sem_i0
        )
        idx_cp1 = pltpu.async_copy(
            idx_hbm.at[pl.ds(base + half, half)], idx_v.at[pl.ds(half, half)], sem_i1
        )
        w_cps = [
            pltpu.async_copy(emb_hbm, e_v, sem_w),
            pltpu.async_copy(w_hbm, w_v, sem_w),
            pltpu.async_copy(b_hbm, b_v, sem_w),
        ]
        for cp in w_cps:
            cp.wait()

        lane = lax.iota(jnp.int32, LANES)
        zero = jnp.zeros((LANES,), jnp.float32)
        zero_i = jnp.zeros((LANES,), jnp.int32)
        row = jnp.minimum(lane >> 2, jnp.full((LANES,), 2, jnp.int32))
        col = lane & jnp.full((LANES,), 3, jnp.int32)
        valid = lane < 12
        ep = jnp.where(valid, plsc.load_gather(e_v, [row, col]), zero)
        wp = jnp.where(valid, plsc.load_gather(w_v, [col, zero_i]), zero)
        bias = plsc.load_gather(b_v, [zero_i])

        scr_v[...] = ep * wp
        seg = lane & jnp.full((LANES,), -4, jnp.int32)
        one_i = jnp.ones((LANES,), jnp.int32)
        sum4 = (
            plsc.load_gather(scr_v, [seg])
            + plsc.load_gather(scr_v, [seg + one_i])
            + plsc.load_gather(scr_v, [seg + 2 * one_i])
            + plsc.load_gather(scr_v, [seg + 3 * one_i])
        )
        one = jnp.ones((LANES,), jnp.float32)
        sig = one / (one + jnp.exp(-(sum4 + bias)))
        scr_v[...] = sig
        table_idx = jnp.minimum(lane * 4, jnp.full((LANES,), 8, jnp.int32))
        scr_v[...] = plsc.load_gather(scr_v, [table_idx])

        def body(j, carry):
            o = j * LANES
            iv = idx_v[pl.ds(o, LANES)]
            out_v[pl.ds(o, LANES)] = plsc.load_gather(scr_v, [iv])
            return carry

        idx_cp0.wait()
        lax.fori_loop(0, half // LANES, body, 0)
        out_cp0 = pltpu.async_copy(
            out_v.at[pl.ds(0, half)], out_hbm.at[pl.ds(base, half)], sem_o0
        )
        idx_cp1.wait()
        lax.fori_loop(half // LANES, b_per_w // LANES, body, 0)
        out_cp1 = pltpu.async_copy(
            out_v.at[pl.ds(half, half)], out_hbm.at[pl.ds(base + half, half)], sem_o1
        )
        out_cp0.wait()
        out_cp1.wait()

    return sc_kernel


def kernel(indices, emb_table, W, b):
    out = _build(indices.shape[0])(indices, emb_table, W, b)
    return out.reshape(-1, 1)


# speedup vs baseline: 1.0359x; 1.0024x over previous
import functools

import jax
import jax.numpy as jnp
from jax import lax
from jax.experimental import pallas as pl
from jax.experimental.pallas import tpu as pltpu
from jax.experimental.pallas import tpu_sc as plsc

LANES = 16


@functools.lru_cache(maxsize=None)
def _build(batch: int):
    info = plsc.get_sparse_core_info()
    nc, ns = 1, info.num_subcores
    nw = nc * ns
    assert batch % (8 * nw) == 0, "HBM 1-D slice offsets must be 8-aligned"
    b_per_w = batch // nw

    mesh = plsc.VectorSubcoreMesh(
        core_axis_name="c", subcore_axis_name="s", num_cores=nc
    )

    @functools.partial(
        pl.kernel,
        mesh=mesh,
        out_type=jax.ShapeDtypeStruct((batch,), jnp.float32),
        compiler_params=pltpu.CompilerParams(
            needs_layout_passes=False, use_tc_tiling_on_sc=False
        ),
        scratch_types=[
            pltpu.VMEM((b_per_w,), jnp.int32),
            pltpu.VMEM((b_per_w,), jnp.float32),
            pltpu.VMEM((3, 4), jnp.float32),
            pltpu.VMEM((4, 1), jnp.float32),
            pltpu.VMEM((1,), jnp.float32),
            pltpu.VMEM((LANES,), jnp.float32),
            pltpu.SemaphoreType.DMA,
            pltpu.SemaphoreType.DMA,
        ],
    )
    def sc_kernel(idx_hbm, emb_hbm, w_hbm, b_hbm, out_hbm, idx_v, out_v, e_v, w_v, b_v, scr_v, sem_i, sem_w):
        wid = lax.axis_index("s") * nc + lax.axis_index("c")
        base = wid * b_per_w
        idx_cp = pltpu.async_copy(idx_hbm.at[pl.ds(base, b_per_w)], idx_v, sem_i)
        w_cps = [
            pltpu.async_copy(emb_hbm, e_v, sem_w),
            pltpu.async_copy(w_hbm, w_v, sem_w),
            pltpu.async_copy(b_hbm, b_v, sem_w),
        ]
        for cp in w_cps:
            cp.wait()

        lane = lax.iota(jnp.int32, LANES)
        zero = jnp.zeros((LANES,), jnp.float32)
        zero_i = jnp.zeros((LANES,), jnp.int32)
        row = jnp.minimum(lane >> 2, jnp.full((LANES,), 2, jnp.int32))
        col = lane & jnp.full((LANES,), 3, jnp.int32)
        valid = lane < 12
        ep = jnp.where(valid, plsc.load_gather(e_v, [row, col]), zero)
        wp = jnp.where(valid, plsc.load_gather(w_v, [col, zero_i]), zero)
        bias = plsc.load_gather(b_v, [zero_i])

        scr_v[...] = ep * wp
        seg = lane & jnp.full((LANES,), -4, jnp.int32)
        one_i = jnp.ones((LANES,), jnp.int32)
        sum4 = (
            plsc.load_gather(scr_v, [seg])
            + plsc.load_gather(scr_v, [seg + one_i])
            + plsc.load_gather(scr_v, [seg + 2 * one_i])
            + plsc.load_gather(scr_v, [seg + 3 * one_i])
        )
        one = jnp.ones((LANES,), jnp.float32)
        sig = one / (one + jnp.exp(-(sum4 + bias)))
        scr_v[...] = sig
        table_idx = jnp.minimum(lane * 4, jnp.full((LANES,), 8, jnp.int32))
        scr_v[...] = plsc.load_gather(scr_v, [table_idx])

        def body(j, carry):
            o = j * LANES
            iv = idx_v[pl.ds(o, LANES)]
            out_v[pl.ds(o, LANES)] = plsc.load_gather(scr_v, [iv])
            return carry

        idx_cp.wait()
        lax.fori_loop(0, b_per_w // LANES, body, 0)
        pltpu.sync_copy(out_v, out_hbm.at[pl.ds(base, b_per_w)])

    return sc_kernel


def kernel(indices, emb_table, W, b):
    out = _build(indices.shape[0])(indices, emb_table, W, b)
    return out.reshape(-1, 1)
